# Initial kernel scaffold; baseline (speedup 1.0000x reference)
import jax
import jax.numpy as jnp
from jax.experimental import pallas as pl


def kernel(x, pos_table):
    raise NotImplementedError("write your pallas kernel here")


# speedup vs baseline: 1.4838x; 1.4838x over previous
import jax
import jax.numpy as jnp
from jax.experimental import pallas as pl

CONTEXT_LENGTH = 8192
EMBEDDING_DIM = 1024
BATCH = 4
SEQ_BLOCK = 512


def _add_kernel(x_ref, pos_ref, out_ref):
    out_ref[...] = x_ref[...] + pos_ref[...][None]


def kernel(x, pos_table):
    grid = (CONTEXT_LENGTH // SEQ_BLOCK, BATCH)
    return pl.pallas_call(
        _add_kernel,
        grid=grid,
        in_specs=[
            pl.BlockSpec((1, SEQ_BLOCK, EMBEDDING_DIM), lambda i, b: (b, i, 0)),
            pl.BlockSpec((SEQ_BLOCK, EMBEDDING_DIM), lambda i, b: (i, 0)),
        ],
        out_specs=pl.BlockSpec((1, SEQ_BLOCK, EMBEDDING_DIM), lambda i, b: (b, i, 0)),
        out_shape=jax.ShapeDtypeStruct(x.shape, x.dtype),
    )(x, pos_table)


# speedup vs baseline: 1.6656x; 1.1226x over previous
import jax
import jax.numpy as jnp
from jax.experimental import pallas as pl

CONTEXT_LENGTH = 8192
EMBEDDING_DIM = 1024
BATCH = 4
SEQ_BLOCK = 1024


def _add_kernel(x_ref, pos_ref, out_ref):
    out_ref[...] = x_ref[...] + pos_ref[...][None]


def kernel(x, pos_table):
    grid = (CONTEXT_LENGTH // SEQ_BLOCK, BATCH)
    return pl.pallas_call(
        _add_kernel,
        grid=grid,
        in_specs=[
            pl.BlockSpec((1, SEQ_BLOCK, EMBEDDING_DIM), lambda i, b: (b, i, 0)),
            pl.BlockSpec((SEQ_BLOCK, EMBEDDING_DIM), lambda i, b: (i, 0)),
        ],
        out_specs=pl.BlockSpec((1, SEQ_BLOCK, EMBEDDING_DIM), lambda i, b: (b, i, 0)),
        out_shape=jax.ShapeDtypeStruct(x.shape, x.dtype),
    )(x, pos_table)


# speedup vs baseline: 1.7387x; 1.0439x over previous
import jax
import jax.numpy as jnp
from jax.experimental import pallas as pl

CONTEXT_LENGTH = 8192
EMBEDDING_DIM = 1024
BATCH = 4
SEQ_BLOCK = 2048


def _add_kernel(x_ref, pos_ref, out_ref):
    out_ref[...] = x_ref[...] + pos_ref[...][None]


def kernel(x, pos_table):
    grid = (CONTEXT_LENGTH // SEQ_BLOCK, BATCH)
    return pl.pallas_call(
        _add_kernel,
        grid=grid,
        in_specs=[
            pl.BlockSpec((1, SEQ_BLOCK, EMBEDDING_DIM), lambda i, b: (b, i, 0)),
            pl.BlockSpec((SEQ_BLOCK, EMBEDDING_DIM), lambda i, b: (i, 0)),
        ],
        out_specs=pl.BlockSpec((1, SEQ_BLOCK, EMBEDDING_DIM), lambda i, b: (b, i, 0)),
        out_shape=jax.ShapeDtypeStruct(x.shape, x.dtype),
    )(x, pos_table)
